# Initial kernel scaffold; baseline (speedup 1.0000x reference)
import jax
import jax.numpy as jnp
from jax.experimental import pallas as pl


def kernel(ss, ps, os, E, R):
    raise NotImplementedError("write your pallas kernel here")


# speedup vs baseline: 2.8265x; 2.8265x over previous
import functools

import numpy as np
import jax
import jax.numpy as jnp
from jax import lax
from jax.experimental import pallas as pl
from jax.experimental.pallas import tpu as pltpu
from jax.experimental.pallas import tpu_sc as plsc

_D = 128
_B = 16384
_NREL_PAD = 128

_ii = np.arange(_D, dtype=np.float64)[:, None]
_jj = np.arange(_D, dtype=np.float64)[None, :]
_G_np = np.where(_jj <= 64,
                 np.cos(2 * np.pi * _jj * _ii / _D),
                 -np.sin(2 * np.pi * (_jj - 64) * _ii / _D))
_GS_np = np.concatenate([_G_np[:, 64:], _G_np[:, :64]], axis=1)

_M1_np = np.zeros((_D, _D))
_M2_np = np.zeros((_D, _D))
for _c in range(_D):
    if _c <= 63:
        _w = (1.0 if _c == 0 else 2.0) / _D
        _M1_np[:, _c] = _w * np.cos(2 * np.pi * _c * _ii[:, 0] / _D)
        if _c > 0:
            _M2_np[:, _c] = -(2.0 / _D) * np.sin(2 * np.pi * _c * _ii[:, 0] / _D)
    elif _c == 64:
        _M1_np[:, _c] = (1.0 / _D) * np.cos(np.pi * _ii[:, 0])
    else:
        _f = _c - 64
        _M1_np[:, _c] = (2.0 / _D) * np.cos(2 * np.pi * _f * _ii[:, 0] / _D)
        _M2_np[:, _c] = (2.0 / _D) * np.sin(2 * np.pi * _f * _ii[:, 0] / _D)

_G = jnp.asarray(_G_np, dtype=jnp.float32)
_GS = jnp.asarray(_GS_np, dtype=jnp.float32)
_M1T = jnp.asarray(_M1_np.T, dtype=jnp.float32)
_M2T = jnp.asarray(_M2_np.T, dtype=jnp.float32)

_NC, _NS = 2, 16
_NW = _NC * _NS
_BPW = _B // _NW
_CH = 128
_NCHUNK = _BPW // _CH

@functools.cache
def _sc_gather_fn():
    mesh = plsc.VectorSubcoreMesh(core_axis_name="c", subcore_axis_name="s")

    @functools.partial(
        pl.kernel,
        out_type=[jax.ShapeDtypeStruct((_B, _D), jnp.float32),
                  jax.ShapeDtypeStruct((_B, _D), jnp.float32)],
        mesh=mesh,
        scratch_types=[
            pltpu.VMEM((_CH,), jnp.int32),
            pltpu.VMEM((_CH, _D), jnp.float32),
            pltpu.SemaphoreType.DMA,
        ],
    )
    def _sc_gather(ss_hbm, os_hbm, e_hbm, es_out, eo_out, idx_v, rows_v, sem):
        wid = lax.axis_index("s") * _NC + lax.axis_index("c")
        base = wid * _BPW
        for src, dst in ((ss_hbm, es_out), (os_hbm, eo_out)):
            for c in range(_NCHUNK):
                off = base + c * _CH
                pltpu.sync_copy(src.at[pl.ds(off, _CH)], idx_v)
                pltpu.async_copy(e_hbm.at[idx_v], rows_v, sem).wait()
                pltpu.sync_copy(rows_v, dst.at[pl.ds(off, _CH)])

    return _sc_gather


_BLK = 512
_GRID = _B // _BLK


def _dot(a, b):
    return jnp.dot(a, b, preferred_element_type=jnp.float32,
                   precision=lax.Precision.HIGHEST)


def _tc_body(es_ref, eo_ref, ps_ref, rt_ref, g_ref, gs_ref, m1t_ref, m2t_ref,
             out_ref, c1_s, c2_s):
    @pl.when(pl.program_id(0) == 0)
    def _prep():
        c1_s[...] = _dot(m1t_ref[...], rt_ref[...])
        c2_s[...] = _dot(m2t_ref[...], rt_ref[...])

    a = _dot(es_ref[...], g_ref[...])
    b = _dot(eo_ref[...], g_ref[...])
    bs = _dot(eo_ref[...], gs_ref[...])
    s = _dot(a * b, c1_s[...]) + _dot(a * bs, c2_s[...])
    msk = ps_ref[...][:, None] == lax.broadcasted_iota(jnp.int32,
                                                       (_BLK, _NREL_PAD), 1)
    out_ref[...] = jnp.sum(jnp.where(msk, s, 0.0), axis=1)


def _tc_score_call(es, eo, ps, rt):
    return pl.pallas_call(
        _tc_body,
        grid=(_GRID,),
        in_specs=[
            pl.BlockSpec((_BLK, _D), lambda i: (i, 0)),
            pl.BlockSpec((_BLK, _D), lambda i: (i, 0)),
            pl.BlockSpec((_BLK,), lambda i: (i,)),
            pl.BlockSpec((_D, _NREL_PAD), lambda i: (0, 0)),
            pl.BlockSpec((_D, _D), lambda i: (0, 0)),
            pl.BlockSpec((_D, _D), lambda i: (0, 0)),
            pl.BlockSpec((_D, _D), lambda i: (0, 0)),
            pl.BlockSpec((_D, _D), lambda i: (0, 0)),
        ],
        out_specs=pl.BlockSpec((_BLK,), lambda i: (i,)),
        out_shape=jax.ShapeDtypeStruct((_B,), jnp.float32),
        scratch_shapes=[
            pltpu.VMEM((_D, _NREL_PAD), jnp.float32),
            pltpu.VMEM((_D, _NREL_PAD), jnp.float32),
        ],
    )(es, eo, ps, rt, _G, _GS, _M1T, _M2T)


def kernel(ss, ps, os, E, R):
    ss = ss.astype(jnp.int32)
    os = os.astype(jnp.int32)
    ps = ps.astype(jnp.int32)
    es, eo = _sc_gather_fn()(ss, os, E)
    rt = jnp.pad(R.T, ((0, 0), (0, _NREL_PAD - R.shape[0])))
    return _tc_score_call(es, eo, ps, rt)


# speedup vs baseline: 4.6316x; 1.6386x over previous
import functools

import numpy as np
import jax
import jax.numpy as jnp
from jax import lax
from jax.experimental import pallas as pl
from jax.experimental.pallas import tpu as pltpu
from jax.experimental.pallas import tpu_sc as plsc

_D = 128
_B = 16384
_NREL_PAD = 128

_ii = np.arange(_D, dtype=np.float64)[:, None]
_jj = np.arange(_D, dtype=np.float64)[None, :]
_G_np = np.where(_jj <= 64,
                 np.cos(2 * np.pi * _jj * _ii / _D),
                 -np.sin(2 * np.pi * (_jj - 64) * _ii / _D))
_GS_np = np.concatenate([_G_np[:, 64:], _G_np[:, :64]], axis=1)

_M1_np = np.zeros((_D, _D))
_M2_np = np.zeros((_D, _D))
for _c in range(_D):
    if _c <= 63:
        _w = (1.0 if _c == 0 else 2.0) / _D
        _M1_np[:, _c] = _w * np.cos(2 * np.pi * _c * _ii[:, 0] / _D)
        if _c > 0:
            _M2_np[:, _c] = -(2.0 / _D) * np.sin(2 * np.pi * _c * _ii[:, 0] / _D)
    elif _c == 64:
        _M1_np[:, _c] = (1.0 / _D) * np.cos(np.pi * _ii[:, 0])
    else:
        _f = _c - 64
        _M1_np[:, _c] = (2.0 / _D) * np.cos(2 * np.pi * _f * _ii[:, 0] / _D)
        _M2_np[:, _c] = (2.0 / _D) * np.sin(2 * np.pi * _f * _ii[:, 0] / _D)

_G = jnp.asarray(_G_np, dtype=jnp.float32)
_M12T = jnp.asarray(np.concatenate([_M1_np.T, _M2_np.T], axis=0),
                    dtype=jnp.float32)

_NC, _NS = 2, 16
_NW = _NC * _NS
_BPW = _B // _NW
_CH = 128
_NCHUNK = _BPW // _CH

@functools.cache
def _sc_gather_fn():
    mesh = plsc.VectorSubcoreMesh(core_axis_name="c", subcore_axis_name="s")

    @functools.partial(
        pl.kernel,
        out_type=[jax.ShapeDtypeStruct((_B, _D), jnp.float32),
                  jax.ShapeDtypeStruct((_B, _D), jnp.float32)],
        mesh=mesh,
        scratch_types=[
            pltpu.VMEM((_CH,), jnp.int32),
            pltpu.VMEM((_CH, _D), jnp.float32),
            pltpu.SemaphoreType.DMA,
        ],
    )
    def _sc_gather(ss_hbm, os_hbm, e_hbm, es_out, eo_out, idx_v, rows_v, sem):
        wid = lax.axis_index("s") * _NC + lax.axis_index("c")
        base = wid * _BPW
        for src, dst in ((ss_hbm, es_out), (os_hbm, eo_out)):
            for c in range(_NCHUNK):
                off = base + c * _CH
                pltpu.sync_copy(src.at[pl.ds(off, _CH)], idx_v)
                pltpu.async_copy(e_hbm.at[idx_v], rows_v, sem).wait()
                pltpu.sync_copy(rows_v, dst.at[pl.ds(off, _CH)])

    return _sc_gather


_BLK = 512
_GRID = _B // _BLK


def _dot(a, b):
    return jnp.dot(a, b, preferred_element_type=jnp.float32,
                   precision=lax.Precision.DEFAULT)


def _tc_body(es_ref, eo_ref, ps_ref, rt_ref, g_ref, m12t_ref,
             out_ref, cc_s):
    @pl.when(pl.program_id(0) == 0)
    def _prep():
        cc_s[...] = _dot(m12t_ref[...], rt_ref[...])

    a = _dot(es_ref[...], g_ref[...])
    b = _dot(eo_ref[...], g_ref[...])
    bs = pltpu.roll(b, 64, axis=1)
    u = jnp.concatenate([a * b, a * bs], axis=1)
    s = _dot(u, cc_s[...])
    msk = ps_ref[...][:, None] == lax.broadcasted_iota(jnp.int32,
                                                       (_BLK, _NREL_PAD), 1)
    out_ref[...] = jnp.sum(jnp.where(msk, s, 0.0), axis=1)


def _tc_score_call(es, eo, ps, rt):
    return pl.pallas_call(
        _tc_body,
        grid=(_GRID,),
        in_specs=[
            pl.BlockSpec((_BLK, _D), lambda i: (i, 0)),
            pl.BlockSpec((_BLK, _D), lambda i: (i, 0)),
            pl.BlockSpec((_BLK,), lambda i: (i,)),
            pl.BlockSpec((_D, _NREL_PAD), lambda i: (0, 0)),
            pl.BlockSpec((_D, _D), lambda i: (0, 0)),
            pl.BlockSpec((2 * _D, _D), lambda i: (0, 0)),
        ],
        out_specs=pl.BlockSpec((_BLK,), lambda i: (i,)),
        out_shape=jax.ShapeDtypeStruct((_B,), jnp.float32),
        scratch_shapes=[
            pltpu.VMEM((2 * _D, _NREL_PAD), jnp.float32),
        ],
    )(es, eo, ps, rt, _G, _M12T)


def kernel(ss, ps, os, E, R):
    ss = ss.astype(jnp.int32)
    os = os.astype(jnp.int32)
    ps = ps.astype(jnp.int32)
    es, eo = _sc_gather_fn()(ss, os, E)
    rt = jnp.pad(R.T, ((0, 0), (0, _NREL_PAD - R.shape[0])))
    return _tc_score_call(es, eo, ps, rt)


# speedup vs baseline: 6.8551x; 1.4801x over previous
import functools

import numpy as np
import jax
import jax.numpy as jnp
from jax import lax
from jax.experimental import pallas as pl
from jax.experimental.pallas import tpu as pltpu
from jax.experimental.pallas import tpu_sc as plsc

_D = 128
_B = 16384
_NREL_PAD = 128

_ii = np.arange(_D, dtype=np.float64)[:, None]
_jj = np.arange(_D, dtype=np.float64)[None, :]
_G_np = np.where(_jj <= 64,
                 np.cos(2 * np.pi * _jj * _ii / _D),
                 -np.sin(2 * np.pi * (_jj - 64) * _ii / _D))
_GS_np = np.concatenate([_G_np[:, 64:], _G_np[:, :64]], axis=1)

_M1_np = np.zeros((_D, _D))
_M2_np = np.zeros((_D, _D))
for _c in range(_D):
    if _c <= 63:
        _w = (1.0 if _c == 0 else 2.0) / _D
        _M1_np[:, _c] = _w * np.cos(2 * np.pi * _c * _ii[:, 0] / _D)
        if _c > 0:
            _M2_np[:, _c] = -(2.0 / _D) * np.sin(2 * np.pi * _c * _ii[:, 0] / _D)
    elif _c == 64:
        _M1_np[:, _c] = (1.0 / _D) * np.cos(np.pi * _ii[:, 0])
    else:
        _f = _c - 64
        _M1_np[:, _c] = (2.0 / _D) * np.cos(2 * np.pi * _f * _ii[:, 0] / _D)
        _M2_np[:, _c] = (2.0 / _D) * np.sin(2 * np.pi * _f * _ii[:, 0] / _D)

_G = jnp.asarray(_G_np, dtype=jnp.float32)
_M12T = jnp.asarray(np.concatenate([_M1_np.T, _M2_np.T], axis=0),
                    dtype=jnp.float32)

_NC, _NS = 2, 16
_NW = _NC * _NS
_BPW = _B // _NW
_CH = 128
_NCHUNK = _BPW // _CH

_NBUF = 3


@functools.cache
def _sc_gather_fn():
    mesh = plsc.VectorSubcoreMesh(core_axis_name="c", subcore_axis_name="s")

    @functools.partial(
        pl.kernel,
        out_type=[jax.ShapeDtypeStruct((_B, _D), jnp.float32),
                  jax.ShapeDtypeStruct((_B, _D), jnp.float32)],
        mesh=mesh,
        scratch_types=[
            pltpu.VMEM((_NCHUNK, _CH), jnp.int32),
            pltpu.VMEM((_NCHUNK, _CH), jnp.int32),
        ] + [pltpu.VMEM((_CH, _D), jnp.float32)] * _NBUF
          + [pltpu.SemaphoreType.DMA] * (2 * _NBUF),
    )
    def _sc_gather(ss_r, os_r, e_hbm, es_out, eo_out, idx_s, idx_o,
                   *bufs_and_sems):
        bufs = bufs_and_sems[:_NBUF]
        gsem = bufs_and_sems[_NBUF:2 * _NBUF]
        wsem = bufs_and_sems[2 * _NBUF:]
        wid = lax.axis_index("s") * _NC + lax.axis_index("c")
        base = wid * _BPW
        pltpu.sync_copy(ss_r.at[wid], idx_s)
        pltpu.sync_copy(os_r.at[wid], idx_o)

        def chunk(c):
            if c < _NCHUNK:
                return idx_s.at[c], es_out.at[pl.ds(base + c * _CH, _CH)]
            return idx_o.at[c - _NCHUNK], eo_out.at[
                pl.ds(base + (c - _NCHUNK) * _CH, _CH)]

        nch = 2 * _NCHUNK
        gd = [None] * _NBUF
        wb = [None] * _NBUF
        for c in range(nch):
            b = c % _NBUF
            if wb[b] is not None:
                wb[b].wait()
            idx, _ = chunk(c)
            gd[b] = pltpu.async_copy(e_hbm.at[idx], bufs[b], gsem[b])
            if c >= 1:
                pb = (c - 1) % _NBUF
                gd[pb].wait()
                _, dst = chunk(c - 1)
                wb[pb] = pltpu.async_copy(bufs[pb], dst, wsem[pb])
        lb = (nch - 1) % _NBUF
        gd[lb].wait()
        _, dst = chunk(nch - 1)
        wb[lb] = pltpu.async_copy(bufs[lb], dst, wsem[lb])
        for b in range(_NBUF):
            if wb[b] is not None:
                wb[b].wait()

    return _sc_gather


_BLK = 2048
_GRID = _B // _BLK


def _dot(a, b):
    return jnp.dot(a, b, preferred_element_type=jnp.float32,
                   precision=lax.Precision.DEFAULT)


def _tc_body(es_ref, eo_ref, ps_ref, rt_ref, g_ref, m12t_ref,
             out_ref, cc_s):
    @pl.when(pl.program_id(0) == 0)
    def _prep():
        cc_s[...] = _dot(m12t_ref[...], rt_ref[...])

    a = _dot(es_ref[...], g_ref[...])
    b = _dot(eo_ref[...], g_ref[...])
    bs = pltpu.roll(b, 64, axis=1)
    u = jnp.concatenate([a * b, a * bs], axis=1)
    s = _dot(u, cc_s[...])
    msk = ps_ref[...][:, None] == lax.broadcasted_iota(jnp.int32,
                                                       (_BLK, _NREL_PAD), 1)
    out_ref[...] = jnp.sum(jnp.where(msk, s, 0.0), axis=1)


def _tc_score_call(es, eo, ps, rt):
    return pl.pallas_call(
        _tc_body,
        grid=(_GRID,),
        in_specs=[
            pl.BlockSpec((_BLK, _D), lambda i: (i, 0)),
            pl.BlockSpec((_BLK, _D), lambda i: (i, 0)),
            pl.BlockSpec((_BLK,), lambda i: (i,)),
            pl.BlockSpec((_D, _NREL_PAD), lambda i: (0, 0)),
            pl.BlockSpec((_D, _D), lambda i: (0, 0)),
            pl.BlockSpec((2 * _D, _D), lambda i: (0, 0)),
        ],
        out_specs=pl.BlockSpec((_BLK,), lambda i: (i,)),
        out_shape=jax.ShapeDtypeStruct((_B,), jnp.float32),
        scratch_shapes=[
            pltpu.VMEM((2 * _D, _NREL_PAD), jnp.float32),
        ],
    )(es, eo, ps, rt, _G, _M12T)


def kernel(ss, ps, os, E, R):
    ss = ss.astype(jnp.int32).reshape(_NW, _NCHUNK, _CH)
    os = os.astype(jnp.int32).reshape(_NW, _NCHUNK, _CH)
    ps = ps.astype(jnp.int32)
    es, eo = _sc_gather_fn()(ss, os, E)
    rt = jnp.pad(R.T, ((0, 0), (0, _NREL_PAD - R.shape[0])))
    return _tc_score_call(es, eo, ps, rt)
